# baseline (speedup 1.0000x reference)
import jax
import jax.numpy as jnp
from jax.experimental import pallas as pl
from jax.experimental.pallas import tpu as pltpu


def _copy_body(src_ref, dst_ref, sem):
    copy = pltpu.make_async_copy(src_ref, dst_ref, sem)
    copy.start()
    copy.wait()


def kernel(x, table):
    del x
    out = pl.pallas_call(
        _copy_body,
        in_specs=[pl.BlockSpec(memory_space=pl.ANY)],
        out_specs=pl.BlockSpec(memory_space=pl.ANY),
        out_shape=jax.ShapeDtypeStruct(table.shape, table.dtype),
        scratch_shapes=[pltpu.SemaphoreType.DMA],
    )(table)
    return out[None]
